# baseline (speedup 1.0000x reference)
import functools
import math

import jax
import jax.numpy as jnp
from jax.experimental import pallas as pl
from jax.experimental.pallas import tpu as pltpu

_DIN = 4096
_H2 = 128
_NE = 64
_TM = 512


def _router_body(x_ref, w1_ref, b1_ref, wg_ref, bg_ref, o_ref):
    xb = x_ref[...].astype(jnp.bfloat16)
    w1b = w1_ref[...].astype(jnp.bfloat16)
    h = jax.lax.dot_general(
        xb, w1b, (((1,), (0,)), ((), ())), preferred_element_type=jnp.float32
    )
    h = h + b1_ref[...]
    h = 0.5 * h * (1.0 + jax.lax.erf(h * (1.0 / math.sqrt(2.0))))
    logits = jax.lax.dot_general(
        h, wg_ref[...], (((1,), (0,)), ((), ())), preferred_element_type=jnp.float32
    )
    logits = logits + bg_ref[...]
    m = jnp.max(logits, axis=-1, keepdims=True)
    e = jnp.exp(logits - m)
    o_ref[...] = e / jnp.sum(e, axis=-1, keepdims=True)


@functools.partial(jax.jit, static_argnames=())
def kernel(x, W1, b1, Wg, bg):
    tokens = x.shape[0]
    grid = (tokens // _TM,)
    out = pl.pallas_call(
        _router_body,
        grid=grid,
        in_specs=[
            pl.BlockSpec((_TM, _DIN), lambda i: (i, 0)),
            pl.BlockSpec((_DIN, _H2), lambda i: (0, 0)),
            pl.BlockSpec((1, _H2), lambda i: (0, 0)),
            pl.BlockSpec((_H2, _NE), lambda i: (0, 0)),
            pl.BlockSpec((1, _NE), lambda i: (0, 0)),
        ],
        out_specs=pl.BlockSpec((_TM, _NE), lambda i: (i, 0)),
        out_shape=jax.ShapeDtypeStruct((tokens, _NE), jnp.float32),
        compiler_params=pltpu.CompilerParams(
            dimension_semantics=("parallel",),
        ),
    )(x, W1, b1.reshape(1, _H2), Wg, bg.reshape(1, _NE))
    return out


# speedup vs baseline: 1.0329x; 1.0329x over previous
import functools
import math

import jax
import jax.numpy as jnp
from jax.experimental import pallas as pl
from jax.experimental.pallas import tpu as pltpu

_DIN = 4096
_H2 = 128
_NE = 64
_TM = 1024


def _router_body(x_ref, w1_ref, b1_ref, wg_ref, bg_ref, o_ref):
    xb = x_ref[...].astype(jnp.bfloat16)
    w1b = w1_ref[...].astype(jnp.bfloat16)
    h = jax.lax.dot_general(
        xb, w1b, (((1,), (0,)), ((), ())), preferred_element_type=jnp.float32
    )
    h = h + b1_ref[...]
    h = 0.5 * h * (1.0 + jax.lax.erf(h * (1.0 / math.sqrt(2.0))))
    logits = jax.lax.dot_general(
        h, wg_ref[...], (((1,), (0,)), ((), ())), preferred_element_type=jnp.float32
    )
    logits = logits + bg_ref[...]
    m = jnp.max(logits, axis=-1, keepdims=True)
    e = jnp.exp(logits - m)
    o_ref[...] = e / jnp.sum(e, axis=-1, keepdims=True)


@functools.partial(jax.jit, static_argnames=())
def kernel(x, W1, b1, Wg, bg):
    tokens = x.shape[0]
    grid = (tokens // _TM,)
    out = pl.pallas_call(
        _router_body,
        grid=grid,
        in_specs=[
            pl.BlockSpec((_TM, _DIN), lambda i: (i, 0)),
            pl.BlockSpec((_DIN, _H2), lambda i: (0, 0)),
            pl.BlockSpec((1, _H2), lambda i: (0, 0)),
            pl.BlockSpec((_H2, _NE), lambda i: (0, 0)),
            pl.BlockSpec((1, _NE), lambda i: (0, 0)),
        ],
        out_specs=pl.BlockSpec((_TM, _NE), lambda i: (i, 0)),
        out_shape=jax.ShapeDtypeStruct((tokens, _NE), jnp.float32),
        compiler_params=pltpu.CompilerParams(
            dimension_semantics=("parallel",),
        ),
    )(x, W1, b1.reshape(1, _H2), Wg, bg.reshape(1, _NE))
    return out
